# baseline (speedup 1.0000x reference)
import jax
import jax.numpy as jnp
from jax import lax
from jax.experimental import pallas as pl
from jax.experimental.pallas import tpu as pltpu
from jax.experimental.pallas import tpu_sc as plsc

_LANES = 16


def _sc_embedding_lookup(table_flat):
    mesh = plsc.VectorSubcoreMesh(core_axis_name="c", subcore_axis_name="s")

    @pl.kernel(
        out_type=jax.ShapeDtypeStruct((2,), jnp.float32),
        mesh=mesh,
        scratch_types=[
            pltpu.VMEM((_LANES,), jnp.int32),
            pltpu.VMEM((_LANES,), jnp.float32),
            pltpu.SemaphoreType.DMA,
        ],
    )
    def body(table_hbm, out_hbm, idx_v, rows_v, sem):
        is_worker0 = jnp.logical_and(
            lax.axis_index("c") == 0, lax.axis_index("s") == 0
        )

        @pl.when(is_worker0)
        def _():
            lane = lax.iota(jnp.int32, _LANES)
            idx_v[...] = jnp.minimum(lane, 1)
            pltpu.async_copy(table_hbm.at[idx_v], rows_v, sem).wait()
            pltpu.sync_copy(rows_v.at[pl.ds(0, 2)], out_hbm)

    return body(table_flat)


def kernel(feature, table):
    del feature
    return _sc_embedding_lookup(table.reshape(2)).reshape(1, 2)


# speedup vs baseline: 1.0666x; 1.0666x over previous
import jax
import jax.numpy as jnp
from jax import lax
from jax.experimental import pallas as pl
from jax.experimental.pallas import tpu as pltpu
from jax.experimental.pallas import tpu_sc as plsc

_LANES = 16


def _sc_embedding_lookup(table_flat):
    mesh = plsc.VectorSubcoreMesh(
        core_axis_name="c", subcore_axis_name="s", num_cores=1, num_subcores=1
    )

    @pl.kernel(
        out_type=jax.ShapeDtypeStruct((2,), jnp.float32),
        mesh=mesh,
        scratch_types=[
            pltpu.VMEM((_LANES,), jnp.float32),
            pltpu.SemaphoreType.DMA,
        ],
    )
    def body(table_hbm, out_hbm, rows_v, sem):
        lane = lax.iota(jnp.int32, _LANES)
        idx = jnp.minimum(lane, 1)
        pltpu.async_copy(table_hbm.at[idx], rows_v, sem).wait()
        pltpu.sync_copy(rows_v.at[pl.ds(0, 2)], out_hbm)

    return body(table_flat)


def kernel(feature, table):
    del feature
    return _sc_embedding_lookup(table.reshape(2)).reshape(1, 2)


# speedup vs baseline: 1.1221x; 1.0520x over previous
import jax
import jax.numpy as jnp
from jax import lax
from jax.experimental import pallas as pl
from jax.experimental.pallas import tpu as pltpu
from jax.experimental.pallas import tpu_sc as plsc


def _sc_embedding_lookup(table):
    mesh = plsc.ScalarSubcoreMesh(axis_name="c", num_cores=1)

    @pl.kernel(
        out_type=jax.ShapeDtypeStruct((2, 1), jnp.float32),
        mesh=mesh,
    )
    def body(table_hbm, out_hbm):
        pltpu.sync_copy(table_hbm, out_hbm)

    return body(table)


def kernel(feature, table):
    del feature
    return _sc_embedding_lookup(table).reshape(1, 2)
